# baseline (speedup 1.0000x reference)
import functools
import jax
import jax.numpy as jnp
from jax import lax
from jax.experimental import pallas as pl
from jax.experimental.pallas import tpu as pltpu
from jax.experimental.pallas import tpu_sc as plsc

B, D, H, E, K, F = 4096, 1024, 128, 8, 2, 1024
N = B * K
BLK = 256
NBLK = (N + E * BLK) // BLK
NS = NBLK * BLK
NT = 16
NW = 32


def _router_body(x_ref, W1_ref, b1_ref, W2_ref, b2_ref, eids_ref, ws_ref):
    x = x_ref[...]
    h1 = jnp.tanh(
        jnp.dot(x, W1_ref[...], preferred_element_type=jnp.float32) + b1_ref[...]
    )
    logits = (
        jnp.dot(h1, W2_ref[...], preferred_element_type=jnp.float32) + b2_ref[...]
    )
    m = jnp.max(logits, axis=1, keepdims=True)
    ex = jnp.exp(logits - m)
    l = ex / jnp.sum(ex, axis=1, keepdims=True)
    iota = lax.broadcasted_iota(jnp.int32, l.shape, 1)
    m1 = jnp.max(l, axis=1, keepdims=True)
    a1 = jnp.min(jnp.where(l == m1, iota, E), axis=1, keepdims=True)
    l2 = jnp.where(iota == a1, -1.0, l)
    m2 = jnp.max(l2, axis=1, keepdims=True)
    a2 = jnp.min(jnp.where(l2 == m2, iota, E), axis=1, keepdims=True)
    t = jnp.exp(m2 - m1)
    s1 = 1.0 / (1.0 + t)
    s2 = t / (1.0 + t)
    eids_ref[...] = jnp.concatenate([a1, a2], axis=1)
    ws_ref[...] = jnp.concatenate([s1, s2], axis=1)


def _dispatch_body(ids_ref, slot_ref, zoff_ref, zlo_ref, blk_ref):
    ids = ids_ref[...]
    r0 = lax.broadcasted_iota(jnp.int32, (128, 128), 0)
    c0 = lax.broadcasted_iota(jnp.int32, (128, 128), 1)
    LT = (r0 <= c0).astype(jnp.float32)
    r1 = lax.broadcasted_iota(jnp.int32, (64, 64), 0)
    c1 = lax.broadcasted_iota(jnp.int32, (64, 64), 1)
    LS = (c1 < r1).astype(jnp.float32)

    row16 = lax.broadcasted_iota(jnp.int32, (16, 16), 0)
    slot = jnp.zeros((64, 128), jnp.int32)
    zoffR = jnp.where(row16 >= 8, N + (row16 - 8) * BLK, 0)
    zloR = jnp.zeros((16, 16), jnp.int32)
    base = jnp.int32(0)
    ends = []
    for e in range(E):
        Mf = (ids == e).astype(jnp.float32)
        Pe = jnp.dot(Mf, LT, preferred_element_type=jnp.float32)
        srow = Pe[:, 127:128]
        T = jnp.dot(LS, srow, preferred_element_type=jnp.float32)
        cnt = jnp.sum(Mf).astype(jnp.int32)
        rank = (Pe + T).astype(jnp.int32)
        slot = jnp.where(ids == e, base + rank - 1, slot)
        aligned = ((cnt + BLK - 1) // BLK) * BLK
        nbase = base + aligned
        zoffR = zoffR + jnp.where(
            row16 == e, jnp.where(cnt > 0, nbase - BLK, NS - BLK), 0
        )
        zloR = zloR + jnp.where(
            row16 == e, jnp.where(cnt < aligned, base + cnt, NS), 0
        )
        ends.append(nbase)
        base = nbase
    zloR = zloR + jnp.where(row16 >= 8, base, 0)
    slot_ref[...] = slot
    zoff_ref[...] = zoffR
    zlo_ref[...] = zloR
    starts = lax.broadcasted_iota(jnp.int32, (1, 64), 1) * BLK
    blk = jnp.zeros((1, 64), jnp.int32)
    for e in range(E):
        blk = blk + (starts >= ends[e]).astype(jnp.int32)
    blk_ref[...] = jnp.minimum(blk, E - 1)


def _gmm_body(bm_ref, xs_ref, we_ref, be_ref, w_ref, yw_ref):
    y = jnp.dot(
        xs_ref[...].astype(jnp.bfloat16),
        we_ref[0],
        preferred_element_type=jnp.float32,
    )
    yw_ref[...] = (y + be_ref[0]) * w_ref[...]


@functools.lru_cache(maxsize=None)
def _sc_scatter_kernel():
    mesh = plsc.VectorSubcoreMesh(core_axis_name="c", subcore_axis_name="s")
    return functools.partial(
        pl.kernel,
        mesh=mesh,
        out_type=[
            jax.ShapeDtypeStruct((NS,), jnp.int32),
            jax.ShapeDtypeStruct((NS,), jnp.float32),
        ],
        scratch_types=[
            pltpu.VMEM((4, 128), jnp.int32),
            pltpu.VMEM((4, 128), jnp.float32),
            pltpu.VMEM((4, 128), jnp.int32),
            pltpu.VMEM((16,), jnp.int32),
            pltpu.VMEM((16,), jnp.int32),
            pltpu.VMEM((2, 128), jnp.int32),
            pltpu.VMEM((2, 128), jnp.int32),
            pltpu.VMEM((2, 128), jnp.float32),
            pltpu.SemaphoreType.DMA,
            pltpu.SemaphoreType.DMA,
        ],
    )(_sc_scatter_body)


def _sc_scatter(slotv3, wv3, zoffR, zloR):
    return _sc_scatter_kernel()(slotv3, wv3, zoffR, zloR)


def _sc_scatter_body(
    slotv_hbm, wv_hbm, zoff_hbm, zlo_hbm, perm_hbm, wsrt_hbm,
    sbuf, wbuf, tokbuf, zobuf, zlbuf, zidx, zi, zf, sem1, sem2,
):
    c = lax.axis_index("c")
    s = lax.axis_index("s")

    @pl.when(c == 0)
    def _():
        lane = lax.iota(jnp.int32, 16)
        pltpu.sync_copy(zoff_hbm.at[s], zobuf)
        pltpu.sync_copy(zlo_hbm.at[s], zlbuf)
        zov = zobuf[...]
        zlv = zlbuf[...]
        for j in range(2):
            for m in range(8):
                raw = zov + j * 128 + m * 16 + lane
                zidx[j, pl.ds(m * 16, 16)] = jnp.minimum(
                    jnp.maximum(raw, zlv), NS - 1
                )
                zi[j, pl.ds(m * 16, 16)] = jnp.zeros((16,), jnp.int32)
                zf[j, pl.ds(m * 16, 16)] = jnp.zeros((16,), jnp.float32)
        zcps = []
        for j in range(2):
            zcps.append(pltpu.async_copy(zi.at[j], perm_hbm.at[zidx.at[j]], sem1))
            zcps.append(pltpu.async_copy(zf.at[j], wsrt_hbm.at[zidx.at[j]], sem2))
        for cp in zcps:
            cp.wait()

        plsc.subcore_barrier()

        pltpu.sync_copy(slotv_hbm.at[s], sbuf)
        pltpu.sync_copy(wv_hbm.at[s], wbuf)
        for j in range(4):
            for m in range(8):
                g0 = s * 512 + j * 128 + m * 16
                tok = lax.shift_right_logical(g0 + lane, 1)
                tokbuf[j, pl.ds(m * 16, 16)] = tok
        cps = []
        for j in range(4):
            cps.append(pltpu.async_copy(tokbuf.at[j], perm_hbm.at[sbuf.at[j]], sem1))
            cps.append(pltpu.async_copy(wbuf.at[j], wsrt_hbm.at[sbuf.at[j]], sem2))
        for cp in cps:
            cp.wait()


@functools.lru_cache(maxsize=None)
def _sc_gather_kernel():
    mesh = plsc.VectorSubcoreMesh(core_axis_name="c", subcore_axis_name="s")
    return functools.partial(
        pl.kernel,
        mesh=mesh,
        out_type=jax.ShapeDtypeStruct((NS, D), jnp.float32),
        scratch_types=[
            pltpu.VMEM((64,), jnp.int32),
            pltpu.VMEM((64, D), jnp.float32),
            pltpu.SemaphoreType.DMA,
        ],
    )(_sc_gather_body)


def _sc_gather(x, perm):
    return _sc_gather_kernel()(x, perm)


def _sc_gather_body(x_hbm, perm_hbm, xs_hbm, idxbuf, rowsbuf, sem):
    c = lax.axis_index("c")
    s = lax.axis_index("s")
    wid = s * 2 + c
    per_w = NS // NW
    for ch in range(per_w // 64):
        off = wid * per_w + ch * 64
        pltpu.sync_copy(perm_hbm.at[pl.ds(off, 64)], idxbuf)
        pltpu.async_copy(x_hbm.at[idxbuf], rowsbuf, sem).wait()
        pltpu.sync_copy(rowsbuf, xs_hbm.at[pl.ds(off, 64)])


@functools.lru_cache(maxsize=None)
def _sc_combine_kernel():
    mesh = plsc.VectorSubcoreMesh(core_axis_name="c", subcore_axis_name="s")
    return functools.partial(
        pl.kernel,
        mesh=mesh,
        out_type=jax.ShapeDtypeStruct((B, F), jnp.float32),
        scratch_types=[
            pltpu.VMEM((64,), jnp.int32),
            pltpu.VMEM((64, F), jnp.float32),
            pltpu.VMEM((32, F), jnp.float32),
            pltpu.SemaphoreType.DMA,
        ],
    )(_sc_combine_body)


def _sc_combine(yw, slotv):
    return _sc_combine_kernel()(yw, slotv)


def _sc_combine_body(yw_hbm, slotv_hbm, out_hbm, idxbuf, prbuf, obuf, sem):
    c = lax.axis_index("c")
    s = lax.axis_index("s")
    wid = s * 2 + c
    tpw = B // NW
    for ch in range(tpw // 32):
        tok0 = wid * tpw + ch * 32
        pltpu.sync_copy(slotv_hbm.at[pl.ds(tok0 * 2, 64)], idxbuf)
        pltpu.async_copy(yw_hbm.at[idxbuf], prbuf, sem).wait()

        def _col(m, _):
            o = m * 16
            for j in range(32):
                a = prbuf[2 * j, pl.ds(o, 16)]
                b = prbuf[2 * j + 1, pl.ds(o, 16)]
                obuf[j, pl.ds(o, 16)] = a + b
            return 0

        lax.fori_loop(0, F // 16, _col, 0)
        pltpu.sync_copy(obuf, out_hbm.at[pl.ds(tok0, 32)])


def kernel(x, W1, b1, W2, b2, We, be):
    nb = 8
    bb = B // nb
    eids, ws = pl.pallas_call(
        _router_body,
        grid=(nb,),
        in_specs=[
            pl.BlockSpec((bb, D), lambda i: (i, 0)),
            pl.BlockSpec((D, H), lambda i: (0, 0)),
            pl.BlockSpec((1, H), lambda i: (0, 0)),
            pl.BlockSpec((H, E), lambda i: (0, 0)),
            pl.BlockSpec((1, E), lambda i: (0, 0)),
        ],
        out_specs=[
            pl.BlockSpec((bb, K), lambda i: (i, 0)),
            pl.BlockSpec((bb, K), lambda i: (i, 0)),
        ],
        out_shape=[
            jax.ShapeDtypeStruct((B, K), jnp.int32),
            jax.ShapeDtypeStruct((B, K), jnp.float32),
        ],
    )(x, W1, b1.reshape(1, H), W2, b2.reshape(1, E))

    slotv, zoffR, zloR, blkmap = pl.pallas_call(
        _dispatch_body,
        grid=(1,),
        in_specs=[pl.BlockSpec((64, 128), lambda i: (0, 0))],
        out_specs=[
            pl.BlockSpec((64, 128), lambda i: (0, 0)),
            pl.BlockSpec((16, 16), lambda i: (0, 0)),
            pl.BlockSpec((16, 16), lambda i: (0, 0)),
            pl.BlockSpec((1, 64), lambda i: (0, 0)),
        ],
        out_shape=[
            jax.ShapeDtypeStruct((64, 128), jnp.int32),
            jax.ShapeDtypeStruct((16, 16), jnp.int32),
            jax.ShapeDtypeStruct((16, 16), jnp.int32),
            jax.ShapeDtypeStruct((1, 64), jnp.int32),
        ],
    )(eids.reshape(64, 128))

    perm, wsrt = _sc_scatter(
        slotv.reshape(NT, 4, 128), ws.reshape(NT, 4, 128), zoffR, zloR
    )
    xs = _sc_gather(x, perm)

    yw = pl.pallas_call(
        _gmm_body,
        grid_spec=pltpu.PrefetchScalarGridSpec(
            num_scalar_prefetch=1,
            grid=(NBLK,),
            in_specs=[
                pl.BlockSpec((BLK, D), lambda i, bm: (i, 0)),
                pl.BlockSpec((1, D, F), lambda i, bm: (bm[i], 0, 0)),
                pl.BlockSpec((1, 1, F), lambda i, bm: (bm[i], 0, 0)),
                pl.BlockSpec((BLK, 1), lambda i, bm: (i, 0)),
            ],
            out_specs=pl.BlockSpec((BLK, F), lambda i, bm: (i, 0)),
        ),
        out_shape=jax.ShapeDtypeStruct((NS, F), jnp.float32),
    )(blkmap.reshape(64), xs, We.astype(jnp.bfloat16), be.reshape(E, 1, F), wsrt.reshape(NS, 1))

    out = _sc_combine(yw, slotv.reshape(N))
    return out


# speedup vs baseline: 2.3661x; 2.3661x over previous
import functools
import jax
import jax.numpy as jnp
from jax import lax
from jax.experimental import pallas as pl
from jax.experimental.pallas import tpu as pltpu
from jax.experimental.pallas import tpu_sc as plsc

B, D, H, E, K, F = 4096, 1024, 128, 8, 2, 1024
N = B * K
BLK = 256
NBLK = (N + E * BLK) // BLK
NS = NBLK * BLK
NW = 32


def _router_body(x_ref, W1_ref, b1_ref, W2_ref, b2_ref, eids_ref, we_ref, wo_ref):
    x = x_ref[...]
    h1 = jnp.tanh(
        jnp.dot(x, W1_ref[...], preferred_element_type=jnp.float32) + b1_ref[...]
    )
    logits = (
        jnp.dot(h1, W2_ref[...], preferred_element_type=jnp.float32) + b2_ref[...]
    )
    m = jnp.max(logits, axis=1, keepdims=True)
    ex = jnp.exp(logits - m)
    l = ex / jnp.sum(ex, axis=1, keepdims=True)
    iota = lax.broadcasted_iota(jnp.int32, l.shape, 1)
    m1 = jnp.max(l, axis=1, keepdims=True)
    a1 = jnp.min(jnp.where(l == m1, iota, E), axis=1, keepdims=True)
    l2 = jnp.where(iota == a1, -1.0, l)
    m2 = jnp.max(l2, axis=1, keepdims=True)
    a2 = jnp.min(jnp.where(l2 == m2, iota, E), axis=1, keepdims=True)
    t = jnp.exp(m2 - m1)
    s1 = 1.0 / (1.0 + t)
    s2 = t / (1.0 + t)
    eids_ref[...] = jnp.concatenate([a1, a2], axis=1)
    ones = jnp.ones((1, 16), jnp.float32)
    we_ref[...] = s1 * ones
    wo_ref[...] = s2 * ones


def _dispatch_body(ids_ref, slot_ref, blk_ref):
    ids = ids_ref[...]
    r0 = lax.broadcasted_iota(jnp.int32, (128, 128), 0)
    c0 = lax.broadcasted_iota(jnp.int32, (128, 128), 1)
    LT = (r0 <= c0).astype(jnp.float32)
    r1 = lax.broadcasted_iota(jnp.int32, (64, 64), 0)
    c1 = lax.broadcasted_iota(jnp.int32, (64, 64), 1)
    LS = (c1 < r1).astype(jnp.float32)

    slot = jnp.zeros((64, 128), jnp.int32)
    base = jnp.int32(0)
    ends = []
    for e in range(E):
        Mf = (ids == e).astype(jnp.float32)
        Pe = jnp.dot(Mf, LT, preferred_element_type=jnp.float32)
        srow = Pe[:, 127:128]
        T = jnp.dot(LS, srow, preferred_element_type=jnp.float32)
        cnt = jnp.sum(Mf).astype(jnp.int32)
        rank = (Pe + T).astype(jnp.int32)
        slot = jnp.where(ids == e, base + rank - 1, slot)
        aligned = ((cnt + BLK - 1) // BLK) * BLK
        nbase = base + aligned
        ends.append(nbase)
        base = nbase
    slot_ref[...] = slot
    starts = lax.broadcasted_iota(jnp.int32, (1, 64), 1) * BLK
    blk = jnp.zeros((1, 64), jnp.int32)
    for e in range(E):
        blk = blk + (starts >= ends[e]).astype(jnp.int32)
    blk_ref[...] = jnp.minimum(blk, E - 1)


def _gmm_body(bm_ref, xs_ref, we_ref, be_ref, yw_ref):
    yw_ref[...] = (
        jnp.dot(
            xs_ref[...].astype(jnp.bfloat16),
            we_ref[0],
            preferred_element_type=jnp.float32,
        )
        + be_ref[0]
    )


@functools.lru_cache(maxsize=None)
def _sc_move_kernel():
    mesh = plsc.VectorSubcoreMesh(core_axis_name="c", subcore_axis_name="s")
    return functools.partial(
        pl.kernel,
        mesh=mesh,
        out_type=jax.ShapeDtypeStruct((NS, D), jnp.float32),
        scratch_types=[
            pltpu.VMEM((64,), jnp.int32),
            pltpu.VMEM((64,), jnp.int32),
            pltpu.VMEM((64, D), jnp.float32),
            pltpu.SemaphoreType.DMA,
            pltpu.SemaphoreType.DMA,
        ],
    )(_sc_move_body)


def _sc_move(x, slotE, slotO):
    return _sc_move_kernel()(x, slotE, slotO)


def _sc_move_body(x_hbm, slotE_hbm, slotO_hbm, xs_hbm, idxE, idxO, xbuf, semE, semO):
    c = lax.axis_index("c")
    s = lax.axis_index("s")
    wid = s * 2 + c
    tpw = B // NW
    for ch in range(tpw // 64):
        tok0 = wid * tpw + ch * 64
        pltpu.sync_copy(slotE_hbm.at[pl.ds(tok0, 64)], idxE)
        pltpu.sync_copy(slotO_hbm.at[pl.ds(tok0, 64)], idxO)
        pltpu.sync_copy(x_hbm.at[pl.ds(tok0, 64)], xbuf)
        cpE = pltpu.async_copy(xbuf, xs_hbm.at[idxE], semE)
        cpO = pltpu.async_copy(xbuf, xs_hbm.at[idxO], semO)
        cpE.wait()
        cpO.wait()


@functools.lru_cache(maxsize=None)
def _sc_combine_kernel():
    mesh = plsc.VectorSubcoreMesh(core_axis_name="c", subcore_axis_name="s")
    return functools.partial(
        pl.kernel,
        mesh=mesh,
        out_type=jax.ShapeDtypeStruct((B, F), jnp.float32),
        scratch_types=[
            pltpu.VMEM((32,), jnp.int32),
            pltpu.VMEM((32,), jnp.int32),
            pltpu.VMEM((32, 16), jnp.float32),
            pltpu.VMEM((32, 16), jnp.float32),
            pltpu.VMEM((32, F), jnp.float32),
            pltpu.VMEM((32, F), jnp.float32),
            pltpu.VMEM((32, F), jnp.float32),
            pltpu.SemaphoreType.DMA,
            pltpu.SemaphoreType.DMA,
        ],
    )(_sc_combine_body)


def _sc_combine(yw, slotE, slotO, wE, wO):
    return _sc_combine_kernel()(yw, slotE, slotO, wE, wO)


def _sc_combine_body(
    yw_hbm, slotE_hbm, slotO_hbm, wE_hbm, wO_hbm, out_hbm,
    idxE, idxO, webuf, wobuf, prE, prO, obuf, semE, semO,
):
    c = lax.axis_index("c")
    s = lax.axis_index("s")
    wid = s * 2 + c
    tpw = B // NW
    for ch in range(tpw // 32):
        tok0 = wid * tpw + ch * 32
        pltpu.sync_copy(slotE_hbm.at[pl.ds(tok0, 32)], idxE)
        pltpu.sync_copy(slotO_hbm.at[pl.ds(tok0, 32)], idxO)
        cpE = pltpu.async_copy(yw_hbm.at[idxE], prE, semE)
        cpO = pltpu.async_copy(yw_hbm.at[idxO], prO, semO)
        pltpu.sync_copy(wE_hbm.at[pl.ds(tok0, 32)], webuf)
        pltpu.sync_copy(wO_hbm.at[pl.ds(tok0, 32)], wobuf)
        cpE.wait()
        cpO.wait()

        def _col(m, _):
            o = m * 16
            for j in range(32):
                obuf[j, pl.ds(o, 16)] = (
                    webuf[j] * prE[j, pl.ds(o, 16)]
                    + wobuf[j] * prO[j, pl.ds(o, 16)]
                )
            return 0

        lax.fori_loop(0, F // 16, _col, 0)
        pltpu.sync_copy(obuf, out_hbm.at[pl.ds(tok0, 32)])


def kernel(x, W1, b1, W2, b2, We, be):
    nb = 8
    bb = B // nb
    eids, wE, wO = pl.pallas_call(
        _router_body,
        grid=(nb,),
        in_specs=[
            pl.BlockSpec((bb, D), lambda i: (i, 0)),
            pl.BlockSpec((D, H), lambda i: (0, 0)),
            pl.BlockSpec((1, H), lambda i: (0, 0)),
            pl.BlockSpec((H, E), lambda i: (0, 0)),
            pl.BlockSpec((1, E), lambda i: (0, 0)),
        ],
        out_specs=[
            pl.BlockSpec((bb, K), lambda i: (i, 0)),
            pl.BlockSpec((bb, 16), lambda i: (i, 0)),
            pl.BlockSpec((bb, 16), lambda i: (i, 0)),
        ],
        out_shape=[
            jax.ShapeDtypeStruct((B, K), jnp.int32),
            jax.ShapeDtypeStruct((B, 16), jnp.float32),
            jax.ShapeDtypeStruct((B, 16), jnp.float32),
        ],
    )(x, W1, b1.reshape(1, H), W2, b2.reshape(1, E))

    slotv, blkmap = pl.pallas_call(
        _dispatch_body,
        grid=(1,),
        in_specs=[pl.BlockSpec((64, 128), lambda i: (0, 0))],
        out_specs=[
            pl.BlockSpec((64, 128), lambda i: (0, 0)),
            pl.BlockSpec((1, 64), lambda i: (0, 0)),
        ],
        out_shape=[
            jax.ShapeDtypeStruct((64, 128), jnp.int32),
            jax.ShapeDtypeStruct((1, 64), jnp.int32),
        ],
    )(eids.reshape(64, 128))

    sl2 = slotv.reshape(B, K)
    slotE = sl2[:, 0]
    slotO = sl2[:, 1]
    xs = _sc_move(x, slotE, slotO)

    yw = pl.pallas_call(
        _gmm_body,
        grid_spec=pltpu.PrefetchScalarGridSpec(
            num_scalar_prefetch=1,
            grid=(NBLK,),
            in_specs=[
                pl.BlockSpec((BLK, D), lambda i, bm: (i, 0)),
                pl.BlockSpec((1, D, F), lambda i, bm: (bm[i], 0, 0)),
                pl.BlockSpec((1, 1, F), lambda i, bm: (bm[i], 0, 0)),
            ],
            out_specs=pl.BlockSpec((BLK, F), lambda i, bm: (i, 0)),
        ),
        out_shape=jax.ShapeDtypeStruct((NS, F), jnp.float32),
    )(blkmap.reshape(64), xs, We.astype(jnp.bfloat16), be.reshape(E, 1, F))

    out = _sc_combine(yw, slotE, slotO, wE, wO)
    return out
